# Initial kernel scaffold; baseline (speedup 1.0000x reference)
import jax
import jax.numpy as jnp
from jax.experimental import pallas as pl


def kernel(node_embeddings, W1, root1, b1, W2, root2, b2, left_in_w, left_in_b, left_out_w, left_out_b, right_in_w, right_in_b, right_out_w, right_out_b, fc1_w, fc1_b, fc2_w, fc2_b, batch, edge_index, edge_type):
    raise NotImplementedError("write your pallas kernel here")


# speedup vs baseline: 3.4850x; 3.4850x over previous
import functools

import jax
import jax.numpy as jnp
import numpy as np
from jax import lax
from jax.experimental import pallas as pl
from jax.experimental.pallas import tpu as pltpu

N = 10000
R = 8
D = 128
H = 8
E = 320000
B = 1024
HD = D // H
LC = 20
LS = 30


def _head_matrix():
    row = lax.broadcasted_iota(jnp.int32, (D, H), 0) // HD
    col = lax.broadcasted_iota(jnp.int32, (D, H), 1)
    return (row == col).astype(jnp.float32)


def _attn_block(q, kv_flat, mask, in_w, in_b, out_w, out_b, L, G):
    BB = q.shape[0]
    Wq = in_w[0:D, :]
    Wk = in_w[D:2 * D, :]
    Wv = in_w[2 * D:3 * D, :]
    bq = in_b[0:D]
    bk = in_b[D:2 * D]
    bv = in_b[2 * D:3 * D]
    qp = jnp.dot(q, Wq.T, preferred_element_type=jnp.float32) + bq[None, :]
    kp = jnp.dot(kv_flat, Wk.T, preferred_element_type=jnp.float32) + bk[None, :]
    vp = jnp.dot(kv_flat, Wv.T, preferred_element_type=jnp.float32) + bv[None, :]
    kp = kp.reshape(BB, L, D)
    vp = vp.reshape(BB, L, D)
    scale = 1.0 / np.sqrt(HD)
    s = []
    for l in range(L):
        sl = jnp.dot(qp * kp[:, l, :], G, preferred_element_type=jnp.float32) * scale
        sl = jnp.where(mask[:, l:l + 1] > 0.5, -1e9, sl)
        s.append(sl)
    m = s[0]
    for l in range(1, L):
        m = jnp.maximum(m, s[l])
    es = [jnp.exp(sl - m) for sl in s]
    den = es[0]
    for l in range(1, L):
        den = den + es[l]
    inv = 1.0 / den
    out = jnp.zeros((BB, D), dtype=jnp.float32)
    for l in range(L):
        a_full = jnp.dot(es[l] * inv, G.T, preferred_element_type=jnp.float32)
        out = out + a_full * vp[:, l, :]
    return jnp.dot(out, out_w.T, preferred_element_type=jnp.float32) + out_b[None, :]


def _batch_stage_kernel(vid_ref, pos_ref, neg_ref, spon_ref, subj_ref,
                        cmask_ref, smask_ref,
                        liw_ref, lib_ref, low_ref, lob_ref,
                        riw_ref, rib_ref, row_ref, rob_ref,
                        f1w_ref, f1b_ref, f2w_ref,
                        pos_out_ref, neg_out_ref):
    BB = vid_ref.shape[0]
    G = _head_matrix()
    left = _attn_block(vid_ref[...], spon_ref[...], cmask_ref[...],
                       liw_ref[...], lib_ref[...], low_ref[...], lob_ref[...], LC, G)
    right = _attn_block(vid_ref[...], subj_ref[...], smask_ref[...],
                        riw_ref[...], rib_ref[...], row_ref[...], rob_ref[...], LS, G)
    f1w = f1w_ref[...]
    f1b = f1b_ref[...]
    f2w = f2w_ref[...]

    def ffnn(leg):
        z = jnp.concatenate([left, right, leg], axis=1)
        h1 = jnp.maximum(jnp.dot(z, f1w.T, preferred_element_type=jnp.float32)
                         + f1b[None, :], 0.0)
        return jnp.sum(h1 * f2w[0][None, :], axis=1)

    pos_out_ref[0, :] = ffnn(pos_ref[...])
    neg_out_ref[0, :] = ffnn(neg_ref[...])


def _batch_stage(vid_e, pos_e, neg_e, spon_e, subj_e, cmask, smask,
                 left_in_w, left_in_b, left_out_w, left_out_b,
                 right_in_w, right_in_b, right_out_w, right_out_b,
                 fc1_w, fc1_b, fc2_w, fc2_b):
    BB = 256
    grid = (B // BB,)
    full = lambda shape: pl.BlockSpec(shape, lambda i: tuple(0 for _ in shape))
    out = pl.pallas_call(
        _batch_stage_kernel,
        grid=grid,
        in_specs=[
            pl.BlockSpec((BB, D), lambda i: (i, 0)),
            pl.BlockSpec((BB, D), lambda i: (i, 0)),
            pl.BlockSpec((BB, D), lambda i: (i, 0)),
            pl.BlockSpec((BB * LC, D), lambda i: (i, 0)),
            pl.BlockSpec((BB * LS, D), lambda i: (i, 0)),
            pl.BlockSpec((BB, LC), lambda i: (i, 0)),
            pl.BlockSpec((BB, LS), lambda i: (i, 0)),
            full((3 * D, D)), full((3 * D,)), full((D, D)), full((D,)),
            full((3 * D, D)), full((3 * D,)), full((D, D)), full((D,)),
            full((3 * D // 2, 3 * D)), full((3 * D // 2,)), full((1, 3 * D // 2)),
        ],
        out_specs=[
            pl.BlockSpec((1, BB), lambda i: (0, i)),
            pl.BlockSpec((1, BB), lambda i: (0, i)),
        ],
        out_shape=[
            jax.ShapeDtypeStruct((1, B), jnp.float32),
            jax.ShapeDtypeStruct((1, B), jnp.float32),
        ],
    )(vid_e, pos_e, neg_e, spon_e, subj_e, cmask, smask,
      left_in_w, left_in_b, left_out_w, left_out_b,
      right_in_w, right_in_b, right_out_w, right_out_b,
      fc1_w, fc1_b, fc2_w)
    return out[0][0] + fc2_b[0], out[1][0] + fc2_b[0]


def kernel(node_embeddings, W1, root1, b1, W2, root2, b2, left_in_w, left_in_b, left_out_w, left_out_b, right_in_w, right_in_b, right_out_w, right_out_b, fc1_w, fc1_b, fc2_w, fc2_b, batch, edge_index, edge_type):
    src, dst = edge_index[0], edge_index[1]
    seg = dst * R + edge_type
    cnt = jax.ops.segment_sum(jnp.ones((E,), dtype=jnp.float32), seg,
                              num_segments=N * R)
    norm = 1.0 / jnp.maximum(cnt, 1.0)

    def rgcn_conv(x, W, Wroot, bb):
        gathered = x[src]
        msum = jax.ops.segment_sum(gathered, seg, num_segments=N * R)
        m = (msum * norm[:, None]).reshape(N, R, D)
        agg = jnp.einsum('nrd,rde->ne', m, W)
        return x @ Wroot + agg + bb

    h = jax.nn.relu(rgcn_conv(node_embeddings, W1, root1, b1))
    ne = rgcn_conv(h, W2, root2, b2)

    vid, pos_idx, neg_idx = batch[:, 0], batch[:, 1], batch[:, 2]
    subj, cosp = batch[:, 3:33], batch[:, 33:]
    cmask = (cosp == 0).astype(jnp.float32)
    smask = (subj == 0).astype(jnp.float32)

    vid_e = ne[vid]
    pos_e = ne[pos_idx]
    neg_e = ne[neg_idx]
    spon_e = ne[cosp].reshape(B * LC, D)
    subj_e = ne[subj].reshape(B * LS, D)

    return _batch_stage(vid_e, pos_e, neg_e, spon_e, subj_e, cmask, smask,
                        left_in_w, left_in_b, left_out_w, left_out_b,
                        right_in_w, right_in_b, right_out_w, right_out_b,
                        fc1_w, fc1_b, fc2_w, fc2_b)


# speedup vs baseline: 15.0377x; 4.3150x over previous
import functools

import jax
import jax.numpy as jnp
import numpy as np
from jax import lax
from jax.experimental import pallas as pl
from jax.experimental.pallas import tpu as pltpu
from jax.experimental.pallas import tpu_sc as plsc

N = 10000
R = 8
D = 128
H = 8
E = 320000
B = 1024
HD = D // H
LC = 20
LS = 30

NC = 2
NS = 16
NW = NC * NS
NROW = 10240
BLK = 512
SEGPAD = 80128
EPW = E // NW
EPWPAD = 10112
KCH = 128
NCHE = EPWPAD // KCH
CH = 1264
NCHC = EPWPAD // CH
EPAD = E + (EPWPAD - EPW)

TRASH_SEG = 80072
TRASH_SRC = N
TRASH_CODE = TRASH_SEG * 16384 + TRASH_SRC

_MESH = dict(core_axis_name="c", subcore_axis_name="s", num_cores=NC,
             num_subcores=NS)


def _cnt_body(src_h, dst_h, ty_h, cnt_out, codes_out, cntv, srcv, dstv, tyv,
              codev):
    c = lax.axis_index("c")
    s = lax.axis_index("s")
    wid = s * NC + c

    def zero(i, _):
        cntv[pl.ds(i * 16, 16)] = jnp.zeros((16,), jnp.float32)
        return 0
    lax.fori_loop(0, SEGPAD // 16, zero, 0)

    base = wid * EPW

    def chunk(i, _):
        off = base + i * CH
        pltpu.sync_copy(src_h.at[pl.ds(off, CH)], srcv)
        pltpu.sync_copy(dst_h.at[pl.ds(off, CH)], dstv)
        pltpu.sync_copy(ty_h.at[pl.ds(off, CH)], tyv)

        def vr(j, _):
            sl = pl.ds(j * 16, 16)
            s16 = srcv[sl]
            seg = dstv[sl] * R + tyv[sl]
            lidx = i * CH + j * 16 + lax.iota(jnp.int32, 16)
            real = lidx < EPW
            code = jnp.where(real, seg * 16384 + s16, TRASH_CODE)
            seg = jnp.where(real, seg, TRASH_SEG)
            codev[sl] = code
            plsc.addupdate_scatter(cntv, [seg], jnp.ones((16,), jnp.float32))
            return 0
        lax.fori_loop(0, CH // 16, vr, 0)
        pltpu.sync_copy(codev, codes_out.at[pl.ds(wid * EPWPAD + i * CH, CH)])
        return 0
    lax.fori_loop(0, NCHC, chunk, 0)
    pltpu.sync_copy(cntv, cnt_out.at[pl.ds(wid * SEGPAD, SEGPAD)])


def _cnt_kernel(srcp, dstp, typ):
    return pl.kernel(
        _cnt_body,
        out_type=(jax.ShapeDtypeStruct((NW * SEGPAD,), jnp.float32),
                  jax.ShapeDtypeStruct((NW * EPWPAD,), jnp.int32)),
        mesh=plsc.VectorSubcoreMesh(**_MESH),
        compiler_params=pltpu.CompilerParams(needs_layout_passes=False),
        scratch_types=[
            pltpu.VMEM((SEGPAD,), jnp.float32),
            pltpu.VMEM((CH,), jnp.int32),
            pltpu.VMEM((CH,), jnp.int32),
            pltpu.VMEM((CH,), jnp.int32),
            pltpu.VMEM((CH,), jnp.int32),
        ],
    )(srcp, dstp, typ)


def _edge_body(xr_h, norm_h, codes_h, part_out, acc,
               cb0, cb1, gi0, gi1, db0, db1, sg0, sg1, nv0, nv1,
               rows0, rows1, sem0, sem1, nsem0, nsem1):
    cb = (cb0, cb1)
    gi = (gi0, gi1)
    db = (db0, db1)
    sg = (sg0, sg1)
    nv = (nv0, nv1)
    rows = (rows0, rows1)
    sems = (sem0, sem1)
    nsems = (nsem0, nsem1)
    c = lax.axis_index("c")
    s = lax.axis_index("s")
    wid = s * NC + c

    def zr(r, _):
        def zm(m, _):
            rows0[r, pl.ds(m * 16, 16)] = jnp.zeros((16,), jnp.float32)
            return 0
        lax.fori_loop(0, D // 16, zm, 0)
        return 0
    lax.fori_loop(0, KCH, zr, 0)

    rows_per_tile = NROW // NS

    def za(k, _):
        pltpu.sync_copy(rows0, acc.at[pl.ds(s * rows_per_tile + k * KCH, KCH)])
        return 0
    lax.fori_loop(0, rows_per_tile // KCH, za, 0)
    plsc.subcore_barrier()

    def load_decode(k, b):
        pltpu.sync_copy(codes_h.at[pl.ds(wid * EPWPAD + k * KCH, KCH)], cb[b])

        def vr(j, _):
            sl = pl.ds(j * 16, 16)
            code = cb[b][sl]
            seg = jnp.right_shift(code, 14)
            srcv = jnp.bitwise_and(code, 16383)
            gi[b][sl] = jnp.bitwise_and(seg, 7) * NROW + srcv
            db[b][sl] = jnp.right_shift(seg, 3)
            sg[b][sl] = seg
            return 0
        lax.fori_loop(0, KCH // 16, vr, 0)

    def issue(b):
        pltpu.async_copy(xr_h.at[gi[b]], rows[b], sems[b])
        pltpu.async_copy(norm_h.at[sg[b]], nv[b], nsems[b])

    def wait(b):
        pltpu.make_async_copy(xr_h.at[gi[b]], rows[b], sems[b]).wait()
        pltpu.make_async_copy(norm_h.at[sg[b]], nv[b], nsems[b]).wait()

    def proc(b):
        def srow(j, _):
            nv16 = nv[b][pl.ds(j * 16, 16)]
            for l in range(16):
                r = j * 16 + l
                scal = jnp.broadcast_to(nv16[l], (16,))
                for m in range(D // 16):
                    sl2 = pl.ds(m * 16, 16)
                    rows[b][r, sl2] = rows[b][r, sl2] * scal
            return 0
        lax.fori_loop(0, KCH // 16, srow, 0)
        pltpu.sync_copy(rows[b], acc.at[db[b]], add=True)

    load_decode(0, 0)
    issue(0)

    def pair(t, _):
        load_decode(2 * t + 1, 1)
        issue(1)
        wait(0)
        proc(0)
        load_decode(2 * t + 2, 0)
        issue(0)
        wait(1)
        proc(1)
        return 0
    lax.fori_loop(0, (NCHE - 1) // 2, pair, 0)
    wait(0)
    proc(0)

    plsc.subcore_barrier()

    def dump(k, _):
        sl = pl.ds(s * rows_per_tile + k * KCH, KCH)
        pltpu.sync_copy(acc.at[sl], part_out.at[c, sl])
        return 0
    lax.fori_loop(0, rows_per_tile // KCH, dump, 0)


def _edge_kernel(xr_flat, norm, codes):
    return pl.kernel(
        _edge_body,
        out_type=jax.ShapeDtypeStruct((NC, NROW, D), jnp.float32),
        mesh=plsc.VectorSubcoreMesh(**_MESH),
        compiler_params=pltpu.CompilerParams(needs_layout_passes=False),
        scratch_types=[
            pltpu.VMEM_SHARED((NROW, D), jnp.float32),
            pltpu.VMEM((KCH,), jnp.int32),
            pltpu.VMEM((KCH,), jnp.int32),
            pltpu.VMEM((KCH,), jnp.int32),
            pltpu.VMEM((KCH,), jnp.int32),
            pltpu.VMEM((KCH,), jnp.int32),
            pltpu.VMEM((KCH,), jnp.int32),
            pltpu.VMEM((KCH,), jnp.int32),
            pltpu.VMEM((KCH,), jnp.int32),
            pltpu.VMEM((KCH,), jnp.float32),
            pltpu.VMEM((KCH,), jnp.float32),
            pltpu.VMEM((KCH, D), jnp.float32),
            pltpu.VMEM((KCH, D), jnp.float32),
            pltpu.SemaphoreType.DMA,
            pltpu.SemaphoreType.DMA,
            pltpu.SemaphoreType.DMA,
            pltpu.SemaphoreType.DMA,
        ],
    )(xr_flat, norm, codes)


def _prep_body(x_ref, w_ref, xr_ref):
    x = x_ref[...]
    for r in range(R):
        xr_ref[r] = jnp.dot(x, w_ref[r], preferred_element_type=jnp.float32)


def _prep(x_pad, W):
    return pl.pallas_call(
        _prep_body,
        grid=(NROW // BLK,),
        in_specs=[
            pl.BlockSpec((BLK, D), lambda i: (i, 0)),
            pl.BlockSpec((R, D, D), lambda i: (0, 0, 0)),
        ],
        out_specs=pl.BlockSpec((R, BLK, D), lambda i: (0, i, 0)),
        out_shape=jax.ShapeDtypeStruct((R, NROW, D), jnp.float32),
    )(x_pad, W)


def _combine1_body(x_ref, p0_ref, p1_ref, root_ref, b_ref, w2_ref,
                   h_ref, xr2_ref):
    h = jnp.dot(x_ref[...], root_ref[...], preferred_element_type=jnp.float32)
    h = jnp.maximum(h + p0_ref[...] + p1_ref[...] + b_ref[...][None, :], 0.0)
    h_ref[...] = h
    for r in range(R):
        xr2_ref[r] = jnp.dot(h, w2_ref[r], preferred_element_type=jnp.float32)


def _combine1(x_pad, p0, p1, root1, b1, W2):
    return pl.pallas_call(
        _combine1_body,
        grid=(NROW // BLK,),
        in_specs=[
            pl.BlockSpec((BLK, D), lambda i: (i, 0)),
            pl.BlockSpec((BLK, D), lambda i: (i, 0)),
            pl.BlockSpec((BLK, D), lambda i: (i, 0)),
            pl.BlockSpec((D, D), lambda i: (0, 0)),
            pl.BlockSpec((D,), lambda i: (0,)),
            pl.BlockSpec((R, D, D), lambda i: (0, 0, 0)),
        ],
        out_specs=[
            pl.BlockSpec((BLK, D), lambda i: (i, 0)),
            pl.BlockSpec((R, BLK, D), lambda i: (0, i, 0)),
        ],
        out_shape=[
            jax.ShapeDtypeStruct((NROW, D), jnp.float32),
            jax.ShapeDtypeStruct((R, NROW, D), jnp.float32),
        ],
    )(x_pad, p0, p1, root1, b1, W2)


def _combine2_body(x_ref, p0_ref, p1_ref, root_ref, b_ref, ne_ref):
    ne = jnp.dot(x_ref[...], root_ref[...], preferred_element_type=jnp.float32)
    ne_ref[...] = ne + p0_ref[...] + p1_ref[...] + b_ref[...][None, :]


def _combine2(h, p0, p1, root2, b2):
    return pl.pallas_call(
        _combine2_body,
        grid=(NROW // BLK,),
        in_specs=[
            pl.BlockSpec((BLK, D), lambda i: (i, 0)),
            pl.BlockSpec((BLK, D), lambda i: (i, 0)),
            pl.BlockSpec((BLK, D), lambda i: (i, 0)),
            pl.BlockSpec((D, D), lambda i: (0, 0)),
            pl.BlockSpec((D,), lambda i: (0,)),
        ],
        out_specs=pl.BlockSpec((BLK, D), lambda i: (i, 0)),
        out_shape=jax.ShapeDtypeStruct((NROW, D), jnp.float32),
    )(h, p0, p1, root2, b2)


def _head_matrix():
    row = lax.broadcasted_iota(jnp.int32, (D, H), 0) // HD
    col = lax.broadcasted_iota(jnp.int32, (D, H), 1)
    return (row == col).astype(jnp.float32)


def _attn_block(q, kv_flat, mask, in_w, in_b, out_w, out_b, L, G):
    BB = q.shape[0]
    Wq = in_w[0:D, :]
    Wk = in_w[D:2 * D, :]
    Wv = in_w[2 * D:3 * D, :]
    bq = in_b[0:D]
    bk = in_b[D:2 * D]
    bv = in_b[2 * D:3 * D]
    qp = jnp.dot(q, Wq.T, preferred_element_type=jnp.float32) + bq[None, :]
    kp = jnp.dot(kv_flat, Wk.T, preferred_element_type=jnp.float32) + bk[None, :]
    vp = jnp.dot(kv_flat, Wv.T, preferred_element_type=jnp.float32) + bv[None, :]
    kp = kp.reshape(BB, L, D)
    vp = vp.reshape(BB, L, D)
    scale = 1.0 / np.sqrt(HD)
    s = []
    for l in range(L):
        sl = jnp.dot(qp * kp[:, l, :], G, preferred_element_type=jnp.float32) * scale
        sl = jnp.where(mask[:, l:l + 1] > 0.5, -1e9, sl)
        s.append(sl)
    m = s[0]
    for l in range(1, L):
        m = jnp.maximum(m, s[l])
    es = [jnp.exp(sl - m) for sl in s]
    den = es[0]
    for l in range(1, L):
        den = den + es[l]
    inv = 1.0 / den
    out = jnp.zeros((BB, D), dtype=jnp.float32)
    for l in range(L):
        a_full = jnp.dot(es[l] * inv, G.T, preferred_element_type=jnp.float32)
        out = out + a_full * vp[:, l, :]
    return jnp.dot(out, out_w.T, preferred_element_type=jnp.float32) + out_b[None, :]


def _batch_stage_kernel(vid_ref, pos_ref, neg_ref, spon_ref, subj_ref,
                        cmask_ref, smask_ref,
                        liw_ref, lib_ref, low_ref, lob_ref,
                        riw_ref, rib_ref, row_ref, rob_ref,
                        f1w_ref, f1b_ref, f2w_ref,
                        pos_out_ref, neg_out_ref):
    G = _head_matrix()
    left = _attn_block(vid_ref[...], spon_ref[...], cmask_ref[...],
                       liw_ref[...], lib_ref[...], low_ref[...], lob_ref[...],
                       LC, G)
    right = _attn_block(vid_ref[...], subj_ref[...], smask_ref[...],
                        riw_ref[...], rib_ref[...], row_ref[...], rob_ref[...],
                        LS, G)
    f1w = f1w_ref[...]
    f1b = f1b_ref[...]
    f2w = f2w_ref[...]

    def ffnn(leg):
        z = jnp.concatenate([left, right, leg], axis=1)
        h1 = jnp.maximum(jnp.dot(z, f1w.T, preferred_element_type=jnp.float32)
                         + f1b[None, :], 0.0)
        return jnp.sum(h1 * f2w[0][None, :], axis=1)

    pos_out_ref[0, :] = ffnn(pos_ref[...])
    neg_out_ref[0, :] = ffnn(neg_ref[...])


def _batch_stage(vid_e, pos_e, neg_e, spon_e, subj_e, cmask, smask,
                 left_in_w, left_in_b, left_out_w, left_out_b,
                 right_in_w, right_in_b, right_out_w, right_out_b,
                 fc1_w, fc1_b, fc2_w, fc2_b):
    BB = 256
    grid = (B // BB,)
    full = lambda shape: pl.BlockSpec(shape, lambda i: tuple(0 for _ in shape))
    out = pl.pallas_call(
        _batch_stage_kernel,
        grid=grid,
        in_specs=[
            pl.BlockSpec((BB, D), lambda i: (i, 0)),
            pl.BlockSpec((BB, D), lambda i: (i, 0)),
            pl.BlockSpec((BB, D), lambda i: (i, 0)),
            pl.BlockSpec((BB * LC, D), lambda i: (i, 0)),
            pl.BlockSpec((BB * LS, D), lambda i: (i, 0)),
            pl.BlockSpec((BB, LC), lambda i: (i, 0)),
            pl.BlockSpec((BB, LS), lambda i: (i, 0)),
            full((3 * D, D)), full((3 * D,)), full((D, D)), full((D,)),
            full((3 * D, D)), full((3 * D,)), full((D, D)), full((D,)),
            full((3 * D // 2, 3 * D)), full((3 * D // 2,)), full((1, 3 * D // 2)),
        ],
        out_specs=[
            pl.BlockSpec((1, BB), lambda i: (0, i)),
            pl.BlockSpec((1, BB), lambda i: (0, i)),
        ],
        out_shape=[
            jax.ShapeDtypeStruct((1, B), jnp.float32),
            jax.ShapeDtypeStruct((1, B), jnp.float32),
        ],
    )(vid_e, pos_e, neg_e, spon_e, subj_e, cmask, smask,
      left_in_w, left_in_b, left_out_w, left_out_b,
      right_in_w, right_in_b, right_out_w, right_out_b,
      fc1_w, fc1_b, fc2_w)
    return out[0][0] + fc2_b[0], out[1][0] + fc2_b[0]


def kernel(node_embeddings, W1, root1, b1, W2, root2, b2, left_in_w, left_in_b, left_out_w, left_out_b, right_in_w, right_in_b, right_out_w, right_out_b, fc1_w, fc1_b, fc2_w, fc2_b, batch, edge_index, edge_type):
    x_pad = jnp.zeros((NROW, D), jnp.float32).at[:N].set(node_embeddings)
    pad_e = EPAD - E
    srcp = jnp.pad(edge_index[0], (0, pad_e))
    dstp = jnp.pad(edge_index[1], (0, pad_e))
    typ = jnp.pad(edge_type, (0, pad_e))

    cnt_all, codes = _cnt_kernel(srcp, dstp, typ)
    cnt = jnp.sum(cnt_all.reshape(NW, SEGPAD), axis=0)
    norm = 1.0 / jnp.maximum(cnt, 1.0)

    xr1 = _prep(x_pad, W1)
    parts1 = _edge_kernel(xr1.reshape(R * NROW, D), norm, codes)
    h, xr2 = _combine1(x_pad, parts1[0], parts1[1], root1, b1, W2)
    parts2 = _edge_kernel(xr2.reshape(R * NROW, D), norm, codes)
    ne = _combine2(h, parts2[0], parts2[1], root2, b2)[:N]

    vid, pos_idx, neg_idx = batch[:, 0], batch[:, 1], batch[:, 2]
    subj, cosp = batch[:, 3:33], batch[:, 33:]
    cmask = (cosp == 0).astype(jnp.float32)
    smask = (subj == 0).astype(jnp.float32)

    vid_e = ne[vid]
    pos_e = ne[pos_idx]
    neg_e = ne[neg_idx]
    spon_e = ne[cosp].reshape(B * LC, D)
    subj_e = ne[subj].reshape(B * LS, D)

    return _batch_stage(vid_e, pos_e, neg_e, spon_e, subj_e, cmask, smask,
                        left_in_w, left_in_b, left_out_w, left_out_b,
                        right_in_w, right_in_b, right_out_w, right_out_b,
                        fc1_w, fc1_b, fc2_w, fc2_b)


# speedup vs baseline: 15.8631x; 1.0549x over previous
import functools

import jax
import jax.numpy as jnp
import numpy as np
from jax import lax
from jax.experimental import pallas as pl
from jax.experimental.pallas import tpu as pltpu
from jax.experimental.pallas import tpu_sc as plsc

N = 10000
R = 8
D = 128
H = 8
E = 320000
B = 1024
HD = D // H
LC = 20
LS = 30

NC = 2
NS = 16
NW = NC * NS
NROW = 10240
BLK = 512
SEGPAD = 80128
EPW = E // NW
EPWPAD = 10112
KCH = 128
NCHE = EPWPAD // KCH
CH = 1264
NCHC = EPWPAD // CH
EPAD = E + (EPWPAD - EPW)

TRASH_SEG = 80072
TRASH_SRC = N
TRASH_CODE = TRASH_SEG * 16384 + TRASH_SRC

_MESH = dict(core_axis_name="c", subcore_axis_name="s", num_cores=NC,
             num_subcores=NS)


def _cnt_body(src_h, dst_h, ty_h, cnt_out, codes_out, cntv, srcv, dstv, tyv,
              codev):
    c = lax.axis_index("c")
    s = lax.axis_index("s")
    wid = s * NC + c

    def zero(i, _):
        cntv[pl.ds(i * 16, 16)] = jnp.zeros((16,), jnp.float32)
        return 0
    lax.fori_loop(0, SEGPAD // 16, zero, 0)

    base = wid * EPW

    def chunk(i, _):
        off = base + i * CH
        pltpu.sync_copy(src_h.at[pl.ds(off, CH)], srcv)
        pltpu.sync_copy(dst_h.at[pl.ds(off, CH)], dstv)
        pltpu.sync_copy(ty_h.at[pl.ds(off, CH)], tyv)

        def vr(j, _):
            sl = pl.ds(j * 16, 16)
            s16 = srcv[sl]
            seg = dstv[sl] * R + tyv[sl]
            lidx = i * CH + j * 16 + lax.iota(jnp.int32, 16)
            real = lidx < EPW
            code = jnp.where(real, seg * 16384 + s16, TRASH_CODE)
            seg = jnp.where(real, seg, TRASH_SEG)
            codev[sl] = code
            plsc.addupdate_scatter(cntv, [seg], jnp.ones((16,), jnp.float32))
            return 0
        lax.fori_loop(0, CH // 16, vr, 0)
        pltpu.sync_copy(codev, codes_out.at[pl.ds(wid * EPWPAD + i * CH, CH)])
        return 0
    lax.fori_loop(0, NCHC, chunk, 0)
    pltpu.sync_copy(cntv, cnt_out.at[pl.ds(wid * SEGPAD, SEGPAD)])


def _cnt_kernel(srcp, dstp, typ):
    return pl.kernel(
        _cnt_body,
        out_type=(jax.ShapeDtypeStruct((NW * SEGPAD,), jnp.float32),
                  jax.ShapeDtypeStruct((NW * EPWPAD,), jnp.int32)),
        mesh=plsc.VectorSubcoreMesh(**_MESH),
        compiler_params=pltpu.CompilerParams(needs_layout_passes=False),
        scratch_types=[
            pltpu.VMEM((SEGPAD,), jnp.float32),
            pltpu.VMEM((CH,), jnp.int32),
            pltpu.VMEM((CH,), jnp.int32),
            pltpu.VMEM((CH,), jnp.int32),
            pltpu.VMEM((CH,), jnp.int32),
        ],
    )(srcp, dstp, typ)


def _edge_body(xr_h, norm_h, codes_h, part_out, acc,
               cb0, cb1, gi0, gi1, db0, db1, sg0, sg1, nv0, nv1,
               rows0, rows1, sem0, sem1, nsem0, nsem1, ssem0, ssem1):
    cb = (cb0, cb1)
    gi = (gi0, gi1)
    db = (db0, db1)
    sg = (sg0, sg1)
    nv = (nv0, nv1)
    rows = (rows0, rows1)
    sems = (sem0, sem1)
    nsems = (nsem0, nsem1)
    ssems = (ssem0, ssem1)
    c = lax.axis_index("c")
    s = lax.axis_index("s")
    wid = s * NC + c

    def zr(r, _):
        def zm(m, _):
            rows0[r, pl.ds(m * 16, 16)] = jnp.zeros((16,), jnp.float32)
            return 0
        lax.fori_loop(0, D // 16, zm, 0)
        return 0
    lax.fori_loop(0, KCH, zr, 0)

    rows_per_tile = NROW // NS

    def za(k, _):
        pltpu.sync_copy(rows0, acc.at[pl.ds(s * rows_per_tile + k * KCH, KCH)])
        return 0
    lax.fori_loop(0, rows_per_tile // KCH, za, 0)
    plsc.subcore_barrier()

    def load_decode(k, b):
        pltpu.sync_copy(codes_h.at[pl.ds(wid * EPWPAD + k * KCH, KCH)], cb[b])

        def vr(j, _):
            sl = pl.ds(j * 16, 16)
            code = cb[b][sl]
            seg = jnp.right_shift(code, 14)
            srcv = jnp.bitwise_and(code, 16383)
            gi[b][sl] = jnp.bitwise_and(seg, 7) * NROW + srcv
            db[b][sl] = jnp.right_shift(seg, 3)
            sg[b][sl] = seg
            return 0
        lax.fori_loop(0, KCH // 16, vr, 0)

    def issue(b):
        pltpu.async_copy(xr_h.at[gi[b]], rows[b], sems[b])
        pltpu.async_copy(norm_h.at[sg[b]], nv[b], nsems[b])

    def wait(b):
        pltpu.make_async_copy(xr_h.at[gi[b]], rows[b], sems[b]).wait()
        pltpu.make_async_copy(norm_h.at[sg[b]], nv[b], nsems[b]).wait()

    def wait_scat(b):
        pltpu.make_async_copy(rows[b], acc.at[db[b]], ssems[b]).wait()

    def proc(b):
        def srow(j, _):
            nv16 = nv[b][pl.ds(j * 16, 16)]
            for l in range(16):
                r = j * 16 + l
                scal = jnp.broadcast_to(nv16[l], (16,))
                for m in range(D // 16):
                    sl2 = pl.ds(m * 16, 16)
                    rows[b][r, sl2] = rows[b][r, sl2] * scal
            return 0
        lax.fori_loop(0, KCH // 16, srow, 0)
        pltpu.async_copy(rows[b], acc.at[db[b]], ssems[b], add=True)

    load_decode(0, 0)
    issue(0)

    def pair(t, _):
        load_decode(2 * t + 1, 1)
        pl.when(t > 0)(lambda: wait_scat(1))
        issue(1)
        wait(0)
        proc(0)
        load_decode(2 * t + 2, 0)
        wait_scat(0)
        issue(0)
        wait(1)
        proc(1)
        return 0
    lax.fori_loop(0, (NCHE - 1) // 2, pair, 0)
    wait(0)
    proc(0)
    wait_scat(1)
    wait_scat(0)

    plsc.subcore_barrier()

    def dump(k, _):
        sl = pl.ds(s * rows_per_tile + k * KCH, KCH)
        pltpu.sync_copy(acc.at[sl], part_out.at[c, sl])
        return 0
    lax.fori_loop(0, rows_per_tile // KCH, dump, 0)


def _edge_kernel(xr_flat, norm, codes):
    return pl.kernel(
        _edge_body,
        out_type=jax.ShapeDtypeStruct((NC, NROW, D), jnp.float32),
        mesh=plsc.VectorSubcoreMesh(**_MESH),
        compiler_params=pltpu.CompilerParams(needs_layout_passes=False),
        scratch_types=[
            pltpu.VMEM_SHARED((NROW, D), jnp.float32),
            pltpu.VMEM((KCH,), jnp.int32),
            pltpu.VMEM((KCH,), jnp.int32),
            pltpu.VMEM((KCH,), jnp.int32),
            pltpu.VMEM((KCH,), jnp.int32),
            pltpu.VMEM((KCH,), jnp.int32),
            pltpu.VMEM((KCH,), jnp.int32),
            pltpu.VMEM((KCH,), jnp.int32),
            pltpu.VMEM((KCH,), jnp.int32),
            pltpu.VMEM((KCH,), jnp.float32),
            pltpu.VMEM((KCH,), jnp.float32),
            pltpu.VMEM((KCH, D), jnp.float32),
            pltpu.VMEM((KCH, D), jnp.float32),
            pltpu.SemaphoreType.DMA,
            pltpu.SemaphoreType.DMA,
            pltpu.SemaphoreType.DMA,
            pltpu.SemaphoreType.DMA,
            pltpu.SemaphoreType.DMA,
            pltpu.SemaphoreType.DMA,
        ],
    )(xr_flat, norm, codes)


def _prep_body(x_ref, w_ref, xr_ref):
    x = x_ref[...]
    for r in range(R):
        xr_ref[r] = jnp.dot(x, w_ref[r], preferred_element_type=jnp.float32)


def _prep(x_pad, W):
    return pl.pallas_call(
        _prep_body,
        grid=(NROW // BLK,),
        in_specs=[
            pl.BlockSpec((BLK, D), lambda i: (i, 0)),
            pl.BlockSpec((R, D, D), lambda i: (0, 0, 0)),
        ],
        out_specs=pl.BlockSpec((R, BLK, D), lambda i: (0, i, 0)),
        out_shape=jax.ShapeDtypeStruct((R, NROW, D), jnp.float32),
    )(x_pad, W)


def _combine1_body(x_ref, p0_ref, p1_ref, root_ref, b_ref, w2_ref,
                   h_ref, xr2_ref):
    h = jnp.dot(x_ref[...], root_ref[...], preferred_element_type=jnp.float32)
    h = jnp.maximum(h + p0_ref[...] + p1_ref[...] + b_ref[...][None, :], 0.0)
    h_ref[...] = h
    for r in range(R):
        xr2_ref[r] = jnp.dot(h, w2_ref[r], preferred_element_type=jnp.float32)


def _combine1(x_pad, p0, p1, root1, b1, W2):
    return pl.pallas_call(
        _combine1_body,
        grid=(NROW // BLK,),
        in_specs=[
            pl.BlockSpec((BLK, D), lambda i: (i, 0)),
            pl.BlockSpec((BLK, D), lambda i: (i, 0)),
            pl.BlockSpec((BLK, D), lambda i: (i, 0)),
            pl.BlockSpec((D, D), lambda i: (0, 0)),
            pl.BlockSpec((D,), lambda i: (0,)),
            pl.BlockSpec((R, D, D), lambda i: (0, 0, 0)),
        ],
        out_specs=[
            pl.BlockSpec((BLK, D), lambda i: (i, 0)),
            pl.BlockSpec((R, BLK, D), lambda i: (0, i, 0)),
        ],
        out_shape=[
            jax.ShapeDtypeStruct((NROW, D), jnp.float32),
            jax.ShapeDtypeStruct((R, NROW, D), jnp.float32),
        ],
    )(x_pad, p0, p1, root1, b1, W2)


def _combine2_body(x_ref, p0_ref, p1_ref, root_ref, b_ref, ne_ref):
    ne = jnp.dot(x_ref[...], root_ref[...], preferred_element_type=jnp.float32)
    ne_ref[...] = ne + p0_ref[...] + p1_ref[...] + b_ref[...][None, :]


def _combine2(h, p0, p1, root2, b2):
    return pl.pallas_call(
        _combine2_body,
        grid=(NROW // BLK,),
        in_specs=[
            pl.BlockSpec((BLK, D), lambda i: (i, 0)),
            pl.BlockSpec((BLK, D), lambda i: (i, 0)),
            pl.BlockSpec((BLK, D), lambda i: (i, 0)),
            pl.BlockSpec((D, D), lambda i: (0, 0)),
            pl.BlockSpec((D,), lambda i: (0,)),
        ],
        out_specs=pl.BlockSpec((BLK, D), lambda i: (i, 0)),
        out_shape=jax.ShapeDtypeStruct((NROW, D), jnp.float32),
    )(h, p0, p1, root2, b2)


def _head_matrix():
    row = lax.broadcasted_iota(jnp.int32, (D, H), 0) // HD
    col = lax.broadcasted_iota(jnp.int32, (D, H), 1)
    return (row == col).astype(jnp.float32)


def _attn_block(q, kv_flat, mask, in_w, in_b, out_w, out_b, L, G):
    BB = q.shape[0]
    Wq = in_w[0:D, :]
    Wk = in_w[D:2 * D, :]
    Wv = in_w[2 * D:3 * D, :]
    bq = in_b[0:D]
    bk = in_b[D:2 * D]
    bv = in_b[2 * D:3 * D]
    qp = jnp.dot(q, Wq.T, preferred_element_type=jnp.float32) + bq[None, :]
    kp = jnp.dot(kv_flat, Wk.T, preferred_element_type=jnp.float32) + bk[None, :]
    vp = jnp.dot(kv_flat, Wv.T, preferred_element_type=jnp.float32) + bv[None, :]
    kp = kp.reshape(BB, L, D)
    vp = vp.reshape(BB, L, D)
    scale = 1.0 / np.sqrt(HD)
    s = []
    for l in range(L):
        sl = jnp.dot(qp * kp[:, l, :], G, preferred_element_type=jnp.float32) * scale
        sl = jnp.where(mask[:, l:l + 1] > 0.5, -1e9, sl)
        s.append(sl)
    m = s[0]
    for l in range(1, L):
        m = jnp.maximum(m, s[l])
    es = [jnp.exp(sl - m) for sl in s]
    den = es[0]
    for l in range(1, L):
        den = den + es[l]
    inv = 1.0 / den
    out = jnp.zeros((BB, D), dtype=jnp.float32)
    for l in range(L):
        a_full = jnp.dot(es[l] * inv, G.T, preferred_element_type=jnp.float32)
        out = out + a_full * vp[:, l, :]
    return jnp.dot(out, out_w.T, preferred_element_type=jnp.float32) + out_b[None, :]


def _batch_stage_kernel(vid_ref, pos_ref, neg_ref, spon_ref, subj_ref,
                        cmask_ref, smask_ref,
                        liw_ref, lib_ref, low_ref, lob_ref,
                        riw_ref, rib_ref, row_ref, rob_ref,
                        f1w_ref, f1b_ref, f2w_ref,
                        pos_out_ref, neg_out_ref):
    G = _head_matrix()
    left = _attn_block(vid_ref[...], spon_ref[...], cmask_ref[...],
                       liw_ref[...], lib_ref[...], low_ref[...], lob_ref[...],
                       LC, G)
    right = _attn_block(vid_ref[...], subj_ref[...], smask_ref[...],
                        riw_ref[...], rib_ref[...], row_ref[...], rob_ref[...],
                        LS, G)
    f1w = f1w_ref[...]
    f1b = f1b_ref[...]
    f2w = f2w_ref[...]

    def ffnn(leg):
        z = jnp.concatenate([left, right, leg], axis=1)
        h1 = jnp.maximum(jnp.dot(z, f1w.T, preferred_element_type=jnp.float32)
                         + f1b[None, :], 0.0)
        return jnp.sum(h1 * f2w[0][None, :], axis=1)

    pos_out_ref[0, :] = ffnn(pos_ref[...])
    neg_out_ref[0, :] = ffnn(neg_ref[...])


def _batch_stage(vid_e, pos_e, neg_e, spon_e, subj_e, cmask, smask,
                 left_in_w, left_in_b, left_out_w, left_out_b,
                 right_in_w, right_in_b, right_out_w, right_out_b,
                 fc1_w, fc1_b, fc2_w, fc2_b):
    BB = 256
    grid = (B // BB,)
    full = lambda shape: pl.BlockSpec(shape, lambda i: tuple(0 for _ in shape))
    out = pl.pallas_call(
        _batch_stage_kernel,
        grid=grid,
        in_specs=[
            pl.BlockSpec((BB, D), lambda i: (i, 0)),
            pl.BlockSpec((BB, D), lambda i: (i, 0)),
            pl.BlockSpec((BB, D), lambda i: (i, 0)),
            pl.BlockSpec((BB * LC, D), lambda i: (i, 0)),
            pl.BlockSpec((BB * LS, D), lambda i: (i, 0)),
            pl.BlockSpec((BB, LC), lambda i: (i, 0)),
            pl.BlockSpec((BB, LS), lambda i: (i, 0)),
            full((3 * D, D)), full((3 * D,)), full((D, D)), full((D,)),
            full((3 * D, D)), full((3 * D,)), full((D, D)), full((D,)),
            full((3 * D // 2, 3 * D)), full((3 * D // 2,)), full((1, 3 * D // 2)),
        ],
        out_specs=[
            pl.BlockSpec((1, BB), lambda i: (0, i)),
            pl.BlockSpec((1, BB), lambda i: (0, i)),
        ],
        out_shape=[
            jax.ShapeDtypeStruct((1, B), jnp.float32),
            jax.ShapeDtypeStruct((1, B), jnp.float32),
        ],
    )(vid_e, pos_e, neg_e, spon_e, subj_e, cmask, smask,
      left_in_w, left_in_b, left_out_w, left_out_b,
      right_in_w, right_in_b, right_out_w, right_out_b,
      fc1_w, fc1_b, fc2_w)
    return out[0][0] + fc2_b[0], out[1][0] + fc2_b[0]


def kernel(node_embeddings, W1, root1, b1, W2, root2, b2, left_in_w, left_in_b, left_out_w, left_out_b, right_in_w, right_in_b, right_out_w, right_out_b, fc1_w, fc1_b, fc2_w, fc2_b, batch, edge_index, edge_type):
    x_pad = jnp.zeros((NROW, D), jnp.float32).at[:N].set(node_embeddings)
    pad_e = EPAD - E
    srcp = jnp.pad(edge_index[0], (0, pad_e))
    dstp = jnp.pad(edge_index[1], (0, pad_e))
    typ = jnp.pad(edge_type, (0, pad_e))

    cnt_all, codes = _cnt_kernel(srcp, dstp, typ)
    cnt = jnp.sum(cnt_all.reshape(NW, SEGPAD), axis=0)
    norm = 1.0 / jnp.maximum(cnt, 1.0)

    xr1 = _prep(x_pad, W1)
    parts1 = _edge_kernel(xr1.reshape(R * NROW, D), norm, codes)
    h, xr2 = _combine1(x_pad, parts1[0], parts1[1], root1, b1, W2)
    parts2 = _edge_kernel(xr2.reshape(R * NROW, D), norm, codes)
    ne = _combine2(h, parts2[0], parts2[1], root2, b2)[:N]

    vid, pos_idx, neg_idx = batch[:, 0], batch[:, 1], batch[:, 2]
    subj, cosp = batch[:, 3:33], batch[:, 33:]
    cmask = (cosp == 0).astype(jnp.float32)
    smask = (subj == 0).astype(jnp.float32)

    vid_e = ne[vid]
    pos_e = ne[pos_idx]
    neg_e = ne[neg_idx]
    spon_e = ne[cosp].reshape(B * LC, D)
    subj_e = ne[subj].reshape(B * LS, D)

    return _batch_stage(vid_e, pos_e, neg_e, spon_e, subj_e, cmask, smask,
                        left_in_w, left_in_b, left_out_w, left_out_b,
                        right_in_w, right_in_b, right_out_w, right_out_b,
                        fc1_w, fc1_b, fc2_w, fc2_b)
